# baseline (speedup 1.0000x reference)
import functools

import jax
import jax.numpy as jnp
from jax import lax
from jax.experimental import pallas as pl
from jax.experimental.pallas import tpu as pltpu
from jax.experimental.pallas import tpu_sc as plsc

NUM_EMB = 1000000
DIM = 32
BATCH = 16384
HIST = 50

NC = 2
NS = 16
NW = NC * NS

_SC_MESH = plsc.VectorSubcoreMesh(core_axis_name="c", subcore_axis_name="s")
_SC_PARAMS = pltpu.CompilerParams(
    use_tc_tiling_on_sc=False, needs_layout_passes=False
)


CR = 512
NCHUNK_R = -(-NUM_EMB // CR)
NCR = -(-NCHUNK_R // NW)
CSUB = 256
assert NCR % 2 == 0


@functools.partial(
    pl.kernel,
    out_type=jax.ShapeDtypeStruct((NUM_EMB, DIM), jnp.float32),
    mesh=_SC_MESH,
    compiler_params=_SC_PARAMS,
    scratch_types=[
        pltpu.VMEM((2, DIM, CR), jnp.float32),
        pltpu.VMEM((2, CR, DIM), jnp.float32),
        pltpu.SemaphoreType.DMA,
        pltpu.SemaphoreType.DMA,
        pltpu.SemaphoreType.DMA,
        pltpu.SemaphoreType.DMA,
    ],
)
def _relayout(wT_hbm, out_hbm, slab, obuf, sl0, sl1, so0, so1):
    wid = lax.axis_index("s") * NC + lax.axis_index("c")
    sem_l = (sl0, sl1)
    sem_o = (so0, so1)
    iota = lax.iota(jnp.int32, 16)

    def chunk_r0(c):
        j = lax.min(wid * NCR + c, NCHUNK_R - 1)
        return lax.min(j * CR, NUM_EMB - CR)

    def slab_desc(c, buf):
        return pltpu.make_async_copy(
            wT_hbm.at[:, pl.ds(chunk_r0(c), CR)], slab.at[buf], sem_l[buf]
        )

    def out_desc(c, buf):
        return pltpu.make_async_copy(
            obuf.at[buf], out_hbm.at[pl.ds(chunk_r0(c), CR), :], sem_o[buf]
        )

    def process(c, buf):
        @pl.when(c + 1 < NCR)
        def _():
            slab_desc(c + 1, 1 - buf).start()

        slab_desc(c, buf).wait()

        @pl.when(c >= 2)
        def _():
            out_desc(c, buf).wait()

        def sub(js, carry):
            r0 = js * CSUB
            for lg in range(CSUB // 16):
                lv = iota + r0 + 16 * lg
                for p in range(DIM):
                    dsw = lax.bitwise_and(iota + p, 31)
                    vals = plsc.load_gather(slab.at[buf], [dsw, lv])
                    plsc.store_scatter(obuf.at[buf], [lv, dsw], vals)
            return carry

        lax.fori_loop(0, CR // CSUB, sub, 0)
        out_desc(c, buf).start()

    slab_desc(0, 0).start()

    def body(i, carry):
        process(2 * i, 0)
        process(2 * i + 1, 1)
        return carry

    lax.fori_loop(0, NCR // 2, body, 0)
    out_desc(NCR - 2, 0).wait()
    out_desc(NCR - 1, 1).wait()


BT = 128
NBT = BATCH // BT
BT_PER_W = NBT // NW
NCHUNK = BT_PER_W * HIST


@functools.partial(
    pl.kernel,
    out_type=jax.ShapeDtypeStruct((HIST, DIM // 8, NBT, 8, BT), jnp.float32),
    mesh=_SC_MESH,
    compiler_params=_SC_PARAMS,
    scratch_types=[
        pltpu.VMEM((HIST, BT_PER_W * BT), jnp.int32),
        pltpu.VMEM((2, BT, DIM), jnp.float32),
        pltpu.VMEM((2, DIM // 8, 8, BT), jnp.float32),
        pltpu.SemaphoreType.DMA,
        pltpu.SemaphoreType.DMA,
        pltpu.SemaphoreType.DMA,
        pltpu.SemaphoreType.DMA,
    ],
)
def _emb_lookup(idxT_hbm, table_hbm, out_hbm, idx_v, lbuf, obuf,
                sg0, sg1, ss0, ss1):
    wid = lax.axis_index("s") * NC + lax.axis_index("c")
    bt0 = wid * BT_PER_W
    pltpu.sync_copy(idxT_hbm.at[:, pl.ds(bt0 * BT, BT_PER_W * BT)], idx_v)
    sem_g = (sg0, sg1)
    sem_s = (ss0, ss1)
    iota = lax.iota(jnp.int32, 16)

    def chunk_hb(c):
        btl = c // HIST
        return btl, c - btl * HIST

    def gather_desc(c, buf):
        btl, h = chunk_hb(c)
        return pltpu.make_async_copy(
            table_hbm.at[idx_v.at[h, pl.ds(btl * BT, BT)]],
            lbuf.at[buf],
            sem_g[buf],
        )

    def scatter_desc(c, buf):
        btl, h = chunk_hb(c)
        return pltpu.make_async_copy(
            obuf.at[buf], out_hbm.at[h, :, bt0 + btl, :, :], sem_s[buf]
        )

    def process(c, buf):
        @pl.when(c + 1 < NCHUNK)
        def _():
            gather_desc(c + 1, 1 - buf).start()

        gather_desc(c, buf).wait()

        @pl.when(c >= 2)
        def _():
            scatter_desc(c, buf).wait()

        for kg in range(BT // 16):
            rowv = iota + 16 * kg
            for p in range(DIM):
                dsw = lax.bitwise_and(iota + p, 31)
                vals = plsc.load_gather(lbuf.at[buf], [rowv, dsw])
                plsc.store_scatter(
                    obuf.at[buf],
                    [lax.shift_right_logical(dsw, 3),
                     lax.bitwise_and(dsw, 7), rowv],
                    vals,
                )

        scatter_desc(c, buf).start()

    gather_desc(0, 0).start()

    def body(i, carry):
        process(2 * i, 0)
        process(2 * i + 1, 1)
        return carry

    lax.fori_loop(0, NCHUNK // 2, body, 0)

    scatter_desc(NCHUNK - 2, 0).wait()
    scatter_desc(NCHUNK - 1, 1).wait()


def kernel(input, weight):
    table = _relayout(weight.T)
    out5 = _emb_lookup(input.T, table)
    return out5.transpose(2, 4, 0, 1, 3).reshape(BATCH, HIST, DIM)


# speedup vs baseline: 4.8553x; 4.8553x over previous
import functools

import jax
import jax.numpy as jnp
from jax import lax
from jax.experimental import pallas as pl
from jax.experimental.pallas import tpu as pltpu
from jax.experimental.pallas import tpu_sc as plsc

NUM_EMB = 1000000
DIM = 32
BATCH = 16384
HIST = 50

NC = 2
NS = 16
NW = NC * NS

BT = 128
NBT = BATCH // BT
BT_PER_W = NBT // NW
NCHUNK = BT_PER_W * HIST
NLINE = NUM_EMB // 4

RELAYOUT_W = 2048


@functools.partial(
    pl.kernel,
    out_type=jax.ShapeDtypeStruct((HIST, DIM // 8, NBT, 8, BT), jnp.float32),
    mesh=plsc.VectorSubcoreMesh(core_axis_name="c", subcore_axis_name="s"),
    compiler_params=pltpu.CompilerParams(
        use_tc_tiling_on_sc=False, needs_layout_passes=False
    ),
    scratch_types=[
        pltpu.VMEM((HIST, BT_PER_W * BT), jnp.int32),
        pltpu.VMEM((2, BT), jnp.int32),
        pltpu.VMEM((2, BT), jnp.int32),
        pltpu.VMEM((2, BT, 128), jnp.float32),
        pltpu.VMEM((2, DIM // 8, 8, BT), jnp.float32),
        pltpu.SemaphoreType.DMA,
        pltpu.SemaphoreType.DMA,
        pltpu.SemaphoreType.DMA,
        pltpu.SemaphoreType.DMA,
    ],
)
def _emb_lookup(idxT_hbm, lines_hbm, out_hbm, idx_v, lines_v, subs_v,
                lbuf, obuf, sg0, sg1, ss0, ss1):
    wid = lax.axis_index("s") * NC + lax.axis_index("c")
    bt0 = wid * BT_PER_W
    pltpu.sync_copy(idxT_hbm.at[:, pl.ds(bt0 * BT, BT_PER_W * BT)], idx_v)
    sem_g = (sg0, sg1)
    sem_s = (ss0, ss1)
    iota = lax.iota(jnp.int32, 16)

    def chunk_hb(c):
        btl = c // HIST
        return btl, c - btl * HIST

    def prep_and_fire(c, buf):
        btl, h = chunk_hb(c)
        for j in range(BT // 16):
            r = idx_v[h, pl.ds(btl * BT + 16 * j, 16)]
            lines_v[buf, pl.ds(16 * j, 16)] = lax.shift_right_logical(r, 2)
            subs_v[buf, pl.ds(16 * j, 16)] = lax.shift_left(
                lax.bitwise_and(r, 3), 5)
        pltpu.make_async_copy(
            lines_hbm.at[lines_v.at[buf]], lbuf.at[buf], sem_g[buf]
        ).start()

    def wait_gather(c, buf):
        pltpu.make_async_copy(
            lines_hbm.at[lines_v.at[buf]], lbuf.at[buf], sem_g[buf]
        ).wait()

    def scatter_desc(c, buf):
        btl, h = chunk_hb(c)
        return pltpu.make_async_copy(
            obuf.at[buf], out_hbm.at[h, :, bt0 + btl, :, :], sem_s[buf]
        )

    def process(c, buf):
        @pl.when(c + 1 < NCHUNK)
        def _():
            prep_and_fire(c + 1, 1 - buf)

        wait_gather(c, buf)

        @pl.when(c >= 2)
        def _():
            scatter_desc(c, buf).wait()

        @functools.partial(plsc.parallel_loop, 0, (BT // 16) * DIM,
                           unroll=8)
        def _(g):
            kg = lax.shift_right_logical(g, 5)
            p = lax.bitwise_and(g, 31)
            rowv = iota + kg * 16
            subv = subs_v[buf, pl.ds(kg * 16, 16)]
            dsw = lax.bitwise_and(iota + p, 31)
            colv = subv + dsw
            vals = plsc.load_gather(lbuf.at[buf], [rowv, colv])
            plsc.store_scatter(
                obuf.at[buf],
                [lax.shift_right_logical(dsw, 3),
                 lax.bitwise_and(dsw, 7), rowv],
                vals,
            )

        scatter_desc(c, buf).start()

    prep_and_fire(0, 0)

    def body(i, carry):
        process(2 * i, 0)
        process(2 * i + 1, 1)
        return carry

    lax.fori_loop(0, NCHUNK // 2, body, 0)

    scatter_desc(NCHUNK - 2, 0).wait()
    scatter_desc(NCHUNK - 1, 1).wait()


def _relayout_body(wt_ref, out_ref):
    t = wt_ref[...].T
    t3 = t.reshape(RELAYOUT_W // 4, 4, DIM)
    out_ref[...] = jnp.concatenate([t3[:, s, :] for s in range(4)], axis=1)


_relayout = pl.pallas_call(
    _relayout_body,
    out_shape=jax.ShapeDtypeStruct((NLINE, 128), jnp.float32),
    grid=((NUM_EMB + RELAYOUT_W - 1) // RELAYOUT_W,),
    in_specs=[pl.BlockSpec((DIM, RELAYOUT_W), lambda i: (0, i))],
    out_specs=pl.BlockSpec((RELAYOUT_W // 4, 128), lambda i: (i, 0)),
)


def kernel(input, weight):
    w128 = _relayout(weight.T)
    out5 = _emb_lookup(input.T, w128)
    return out5.transpose(2, 4, 0, 1, 3).reshape(BATCH, HIST, DIM)
